# Initial kernel scaffold; baseline (speedup 1.0000x reference)
import jax
import jax.numpy as jnp
from jax.experimental import pallas as pl


def kernel(X_batch, batch_idx, W, b):
    raise NotImplementedError("write your pallas kernel here")


# speedup vs baseline: 2.2064x; 2.2064x over previous
import functools

import jax
import jax.numpy as jnp
import numpy as np
from jax import lax
from jax.experimental import pallas as pl
from jax.experimental.pallas import tpu as pltpu

N = 65536
G = 2048
T = 20
S = 256
SCALE = 1000000.0

BN = 512
NBLK = N // BN
C = 128


def _helmert_basis(D):
    H = np.zeros((D - 1, D), dtype=np.float32)
    for i in range(D - 1):
        H[i, : i + 1] = 1.0 / (i + 1)
        H[i, i + 1] = -1.0
        H[i] *= np.sqrt((i + 1) / (i + 2))
    return H


def _dot(a, b):
    return lax.dot_general(a, b, (((1,), (0,)), ((), ())),
                           preferred_element_type=jnp.float32)


def _body(x_ref, idx_ref, w_ref, v_ref, b_ref, o_ref):
    i = pl.program_id(0)
    x = x_ref[...]
    lib = jnp.maximum(jnp.sum(x, axis=1, keepdims=True), 1e-8)
    xn = x * (SCALE / lib)
    t = _dot(xn, w_ref[...]) + b_ref[...]
    logx = _dot(t, v_ref[...])
    lane = lax.broadcasted_iota(jnp.int32, (BN, C), 1)
    logx = jnp.where(lane < T + 1, logx, -jnp.inf)
    m = jnp.max(logx, axis=1, keepdims=True)
    e = jnp.exp(logx - m)
    y = e / jnp.sum(e, axis=1, keepdims=True)

    idx = idx_ref[0, 0, :]
    oh = (lax.broadcasted_iota(jnp.int32, (S, BN), 0)
          == idx[None, :]).astype(jnp.bfloat16)
    yh = y.astype(jnp.bfloat16)
    yl = (y - yh.astype(jnp.float32)).astype(jnp.bfloat16)
    part = _dot(oh, yh) + _dot(oh, yl)

    @pl.when(i == 0)
    def _init():
        o_ref[...] = jnp.zeros_like(o_ref)

    o_ref[...] += part

    @pl.when(i == NBLK - 1)
    def _finish():
        acc = o_ref[...]
        denom = jnp.maximum(jnp.sum(acc, axis=1, keepdims=True), 1e-8)
        o_ref[...] = acc / denom


@jax.jit
def kernel(X_batch, batch_idx, W, b):
    Wp = jnp.pad(W, ((0, 0), (0, C - T)))
    V = jnp.asarray(_helmert_basis(T + 1))
    Vp = jnp.pad(V, ((0, C - T), (0, C - (T + 1))))
    bp = jnp.pad(b, (0, C - T)).reshape(1, C)
    idx3 = batch_idx.astype(jnp.int32).reshape(NBLK, 1, BN)

    out = pl.pallas_call(
        _body,
        grid=(NBLK,),
        in_specs=[
            pl.BlockSpec((BN, G), lambda i: (i, 0)),
            pl.BlockSpec((1, 1, BN), lambda i: (i, 0, 0)),
            pl.BlockSpec((G, C), lambda i: (0, 0)),
            pl.BlockSpec((C, C), lambda i: (0, 0)),
            pl.BlockSpec((1, C), lambda i: (0, 0)),
        ],
        out_specs=pl.BlockSpec((S, C), lambda i: (0, 0)),
        out_shape=jax.ShapeDtypeStruct((S, C), jnp.float32),
        compiler_params=pltpu.CompilerParams(
            dimension_semantics=("arbitrary",),
        ),
    )(X_batch, idx3, Wp, Vp, bp)
    return out[:, : T + 1]


# speedup vs baseline: 2.3229x; 1.0528x over previous
import jax
import jax.numpy as jnp
import numpy as np
from jax import lax
from jax.experimental import pallas as pl
from jax.experimental.pallas import tpu as pltpu

N = 65536
G = 2048
T = 20
S = 256
SCALE = 1000000.0

BN = 512
NBLK = N // BN
C = 128
R = 32


def _helmert_basis(D):
    H = np.zeros((D - 1, D), dtype=np.float32)
    for i in range(D - 1):
        H[i, : i + 1] = 1.0 / (i + 1)
        H[i, i + 1] = -1.0
        H[i] *= np.sqrt((i + 1) / (i + 2))
    return H


def _dot(a, b, dims):
    return lax.dot_general(a, b, (dims, ((), ())),
                           preferred_element_type=jnp.float32)


def _body(x_ref, idx_ref, w_ref, v_ref, b_ref, o_ref):
    i = pl.program_id(0)
    x = x_ref[...]
    lib = jnp.maximum(jnp.sum(x, axis=1, keepdims=True), 1e-8)
    xn = x * (SCALE / lib)
    t = _dot(xn, w_ref[...], (((1,), (0,)))) + b_ref[...]
    logxT = _dot(v_ref[...], t, (((0,), (1,))))
    row = lax.broadcasted_iota(jnp.int32, (R, BN), 0)
    logxT = jnp.where(row < T + 1, logxT, -jnp.inf)
    m = jnp.max(logxT, axis=0, keepdims=True)
    e = jnp.exp(logxT - m)
    y = e / jnp.sum(e, axis=0, keepdims=True)

    idx = idx_ref[0, 0, :]
    oh = (lax.broadcasted_iota(jnp.int32, (S, BN), 0)
          == idx[None, :]).astype(jnp.bfloat16)
    yh = y.astype(jnp.bfloat16)
    yl = (y - yh.astype(jnp.float32)).astype(jnp.bfloat16)
    part = (_dot(yh, oh, (((1,), (1,))))
            + _dot(yl, oh, (((1,), (1,)))))

    @pl.when(i == 0)
    def _init():
        o_ref[...] = jnp.zeros_like(o_ref)

    o_ref[...] += part

    @pl.when(i == NBLK - 1)
    def _finish():
        acc = o_ref[...]
        denom = jnp.maximum(jnp.sum(acc, axis=0, keepdims=True), 1e-8)
        o_ref[...] = acc / denom


@jax.jit
def kernel(X_batch, batch_idx, W, b):
    Wp = jnp.pad(W, ((0, 0), (0, C - T)))
    V = jnp.asarray(_helmert_basis(T + 1))
    Vp = jnp.pad(V, ((0, C - T), (0, R - (T + 1))))
    bp = jnp.pad(b, (0, C - T)).reshape(1, C)
    idx3 = batch_idx.astype(jnp.int32).reshape(NBLK, 1, BN)

    out = pl.pallas_call(
        _body,
        grid=(NBLK,),
        in_specs=[
            pl.BlockSpec((BN, G), lambda i: (i, 0)),
            pl.BlockSpec((1, 1, BN), lambda i: (i, 0, 0)),
            pl.BlockSpec((G, C), lambda i: (0, 0)),
            pl.BlockSpec((C, R), lambda i: (0, 0)),
            pl.BlockSpec((1, C), lambda i: (0, 0)),
        ],
        out_specs=pl.BlockSpec((R, S), lambda i: (0, 0)),
        out_shape=jax.ShapeDtypeStruct((R, S), jnp.float32),
        compiler_params=pltpu.CompilerParams(
            dimension_semantics=("arbitrary",),
        ),
    )(X_batch, idx3, Wp, Vp, bp)
    return out.T[:, : T + 1]


# speedup vs baseline: 2.8388x; 1.2221x over previous
import jax
import jax.numpy as jnp
import numpy as np
from jax import lax
from jax.experimental import pallas as pl
from jax.experimental.pallas import tpu as pltpu

N = 65536
G = 2048
T = 20
S = 256
SCALE = 1000000.0

BN = 1024
NBLK = N // BN
C = 128
R = 32


def _helmert_basis(D):
    H = np.zeros((D - 1, D), dtype=np.float32)
    for i in range(D - 1):
        H[i, : i + 1] = 1.0 / (i + 1)
        H[i, i + 1] = -1.0
        H[i] *= np.sqrt((i + 1) / (i + 2))
    return H


def _dot(a, b, dims):
    return lax.dot_general(a, b, (dims, ((), ())),
                           preferred_element_type=jnp.float32)


def _body(x_ref, idx_ref, w_ref, v_ref, b_ref, o_ref):
    i = pl.program_id(0)
    x = x_ref[...]
    lib = jnp.maximum(jnp.sum(x, axis=1, keepdims=True), 1e-8)
    xn = x * (SCALE / lib)
    t = _dot(xn, w_ref[...], (((1,), (0,)))) + b_ref[...]
    logxT = _dot(v_ref[...], t, (((0,), (1,))))
    row = lax.broadcasted_iota(jnp.int32, (R, BN), 0)
    logxT = jnp.where(row < T + 1, logxT, -jnp.inf)
    m = jnp.max(logxT, axis=0, keepdims=True)
    e = jnp.exp(logxT - m)
    y = e / jnp.sum(e, axis=0, keepdims=True)

    idx = idx_ref[0, 0, :]
    oh = (lax.broadcasted_iota(jnp.int32, (S, BN), 0)
          == idx[None, :]).astype(jnp.bfloat16)
    yh = y.astype(jnp.bfloat16)
    yl = (y - yh.astype(jnp.float32)).astype(jnp.bfloat16)
    part = (_dot(yh, oh, (((1,), (1,))))
            + _dot(yl, oh, (((1,), (1,)))))

    @pl.when(i == 0)
    def _init():
        o_ref[...] = jnp.zeros_like(o_ref)

    o_ref[...] += part

    @pl.when(i == NBLK - 1)
    def _finish():
        acc = o_ref[...]
        denom = jnp.maximum(jnp.sum(acc, axis=0, keepdims=True), 1e-8)
        o_ref[...] = acc / denom


@jax.jit
def kernel(X_batch, batch_idx, W, b):
    Wp = jnp.pad(W, ((0, 0), (0, C - T)))
    V = jnp.asarray(_helmert_basis(T + 1))
    Vp = jnp.pad(V, ((0, C - T), (0, R - (T + 1))))
    bp = jnp.pad(b, (0, C - T)).reshape(1, C)
    idx3 = batch_idx.astype(jnp.int32).reshape(NBLK, 1, BN)

    out = pl.pallas_call(
        _body,
        grid=(NBLK,),
        in_specs=[
            pl.BlockSpec((BN, G), lambda i: (i, 0)),
            pl.BlockSpec((1, 1, BN), lambda i: (i, 0, 0)),
            pl.BlockSpec((G, C), lambda i: (0, 0)),
            pl.BlockSpec((C, R), lambda i: (0, 0)),
            pl.BlockSpec((1, C), lambda i: (0, 0)),
        ],
        out_specs=pl.BlockSpec((R, S), lambda i: (0, 0)),
        out_shape=jax.ShapeDtypeStruct((R, S), jnp.float32),
        compiler_params=pltpu.CompilerParams(
            dimension_semantics=("arbitrary",),
        ),
    )(X_batch, idx3, Wp, Vp, bp)
    return out.T[:, : T + 1]


# speedup vs baseline: 3.1774x; 1.1193x over previous
import jax
import jax.numpy as jnp
import numpy as np
from jax import lax
from jax.experimental import pallas as pl
from jax.experimental.pallas import tpu as pltpu

N = 65536
G = 2048
T = 20
S = 256
SCALE = 1000000.0

BN = 2048
NBLK = N // BN
C = 128
R = 32


def _helmert_basis(D):
    H = np.zeros((D - 1, D), dtype=np.float32)
    for i in range(D - 1):
        H[i, : i + 1] = 1.0 / (i + 1)
        H[i, i + 1] = -1.0
        H[i] *= np.sqrt((i + 1) / (i + 2))
    return H


def _dot(a, b, dims):
    return lax.dot_general(a, b, (dims, ((), ())),
                           preferred_element_type=jnp.float32)


def _body(x_ref, idx_ref, w_ref, v_ref, b_ref, o_ref):
    i = pl.program_id(0)
    x = x_ref[...]
    lib = jnp.maximum(jnp.sum(x, axis=1, keepdims=True), 1e-8)
    xn = x * (SCALE / lib)
    t = _dot(xn, w_ref[...], (((1,), (0,)))) + b_ref[...]
    logxT = _dot(v_ref[...], t, (((0,), (1,))))
    row = lax.broadcasted_iota(jnp.int32, (R, BN), 0)
    logxT = jnp.where(row < T + 1, logxT, -jnp.inf)
    m = jnp.max(logxT, axis=0, keepdims=True)
    e = jnp.exp(logxT - m)
    y = e / jnp.sum(e, axis=0, keepdims=True)

    idx = idx_ref[0, 0, :]
    oh = (lax.broadcasted_iota(jnp.int32, (S, BN), 0)
          == idx[None, :]).astype(jnp.bfloat16)
    yh = y.astype(jnp.bfloat16)
    yl = (y - yh.astype(jnp.float32)).astype(jnp.bfloat16)
    part = (_dot(yh, oh, (((1,), (1,))))
            + _dot(yl, oh, (((1,), (1,)))))

    @pl.when(i == 0)
    def _init():
        o_ref[...] = jnp.zeros_like(o_ref)

    o_ref[...] += part

    @pl.when(i == NBLK - 1)
    def _finish():
        acc = o_ref[...]
        denom = jnp.maximum(jnp.sum(acc, axis=0, keepdims=True), 1e-8)
        o_ref[...] = acc / denom


@jax.jit
def kernel(X_batch, batch_idx, W, b):
    Wp = jnp.pad(W, ((0, 0), (0, C - T)))
    V = jnp.asarray(_helmert_basis(T + 1))
    Vp = jnp.pad(V, ((0, C - T), (0, R - (T + 1))))
    bp = jnp.pad(b, (0, C - T)).reshape(1, C)
    idx3 = batch_idx.astype(jnp.int32).reshape(NBLK, 1, BN)

    out = pl.pallas_call(
        _body,
        grid=(NBLK,),
        in_specs=[
            pl.BlockSpec((BN, G), lambda i: (i, 0)),
            pl.BlockSpec((1, 1, BN), lambda i: (i, 0, 0)),
            pl.BlockSpec((G, C), lambda i: (0, 0)),
            pl.BlockSpec((C, R), lambda i: (0, 0)),
            pl.BlockSpec((1, C), lambda i: (0, 0)),
        ],
        out_specs=pl.BlockSpec((R, S), lambda i: (0, 0)),
        out_shape=jax.ShapeDtypeStruct((R, S), jnp.float32),
        compiler_params=pltpu.CompilerParams(
            dimension_semantics=("arbitrary",),
        ),
    )(X_batch, idx3, Wp, Vp, bp)
    return out.T[:, : T + 1]
